# baseline (speedup 1.0000x reference)
import jax
import jax.numpy as jnp
from jax.experimental import pallas as pl
from jax.experimental.pallas import tpu as pltpu

V = 32
C = 128


def _normalize_block(pc_ref, counts_ref, out_ref, rs_ref):
    x = counts_ref[0]
    pc = pc_ref[0, 0]
    ones_col = jnp.ones((C, 1), jnp.float32)
    colsum = jax.lax.dot(
        x.reshape(V * C, C), ones_col, preferred_element_type=jnp.float32
    )
    rs_col = colsum.reshape(V, C, 1).sum(axis=0) + pc * jnp.float32(V * C)
    denom = jnp.where(rs_col > 0, rs_col, jnp.float32(1.0))
    recip = jnp.float32(1.0) / denom
    out_ref[0] = x * recip[None] + (pc * recip)[None]
    rs_ref[0, 0] = rs_col[:, 0]


def kernel(transition_counts, pseudocount, hidden_states):
    del hidden_states
    counts = transition_counts.reshape(V, V, C, C)
    pc = jnp.asarray(pseudocount, jnp.float32).reshape(1, 1)
    out, rs = pl.pallas_call(
        _normalize_block,
        grid=(V,),
        in_specs=[
            pl.BlockSpec(memory_space=pltpu.SMEM),
            pl.BlockSpec((1, V, C, C), lambda i: (i, 0, 0, 0)),
        ],
        out_specs=[
            pl.BlockSpec((1, V, C, C), lambda i: (i, 0, 0, 0)),
            pl.BlockSpec((1, 1, C), lambda i: (i, 0, 0)),
        ],
        out_shape=[
            jax.ShapeDtypeStruct((V, V, C, C), jnp.float32),
            jax.ShapeDtypeStruct((V, 1, C), jnp.float32),
        ],
        compiler_params=pltpu.CompilerParams(
            dimension_semantics=("parallel",),
        ),
    )(pc, counts)
    return out.reshape(-1), rs.reshape(-1)


# speedup vs baseline: 1.2797x; 1.2797x over previous
import jax
import jax.numpy as jnp
from jax.experimental import pallas as pl
from jax.experimental.pallas import tpu as pltpu

V = 32
C = 128
BS = 2


def _normalize_block(pc_ref, counts_ref, out_ref, rs_ref):
    x = counts_ref[...]
    pc = pc_ref[0, 0]
    rs = x.sum(axis=1).sum(axis=2) + pc * jnp.float32(V * C)
    denom = jnp.where(rs > 0, rs, jnp.float32(1.0))
    recip = jnp.float32(1.0) / denom
    out_ref[...] = x * recip[:, None, :, None] + (pc * recip)[:, None, :, None]
    rs_ref[:, 0, :] = rs


def kernel(transition_counts, pseudocount, hidden_states):
    del hidden_states
    counts = transition_counts.reshape(V, V, C, C)
    pc = jnp.asarray(pseudocount, jnp.float32).reshape(1, 1)
    out, rs = pl.pallas_call(
        _normalize_block,
        grid=(V // BS,),
        in_specs=[
            pl.BlockSpec(memory_space=pltpu.SMEM),
            pl.BlockSpec((BS, V, C, C), lambda i: (i, 0, 0, 0)),
        ],
        out_specs=[
            pl.BlockSpec((BS, V, C, C), lambda i: (i, 0, 0, 0)),
            pl.BlockSpec((BS, 1, C), lambda i: (i, 0, 0)),
        ],
        out_shape=[
            jax.ShapeDtypeStruct((V, V, C, C), jnp.float32),
            jax.ShapeDtypeStruct((V, 1, C), jnp.float32),
        ],
        compiler_params=pltpu.CompilerParams(
            dimension_semantics=("arbitrary",),
        ),
    )(pc, counts)
    return out.reshape(-1), rs.reshape(-1)


# speedup vs baseline: 1.3171x; 1.0293x over previous
import jax
import jax.numpy as jnp
from jax.experimental import pallas as pl
from jax.experimental.pallas import tpu as pltpu

V = 32
C = 128
BS = 4


def _normalize_block(pc_ref, counts_ref, out_ref, rs_ref):
    x = counts_ref[...]
    pc = pc_ref[0, 0]
    rs = x.sum(axis=1).sum(axis=2) + pc * jnp.float32(V * C)
    denom = jnp.where(rs > 0, rs, jnp.float32(1.0))
    recip = jnp.float32(1.0) / denom
    out_ref[...] = x * recip[:, None, :, None] + (pc * recip)[:, None, :, None]
    rs_ref[:, 0, :] = rs


def kernel(transition_counts, pseudocount, hidden_states):
    del hidden_states
    counts = transition_counts.reshape(V, V, C, C)
    pc = jnp.asarray(pseudocount, jnp.float32).reshape(1, 1)
    out, rs = pl.pallas_call(
        _normalize_block,
        grid=(V // BS,),
        in_specs=[
            pl.BlockSpec(memory_space=pltpu.SMEM),
            pl.BlockSpec((BS, V, C, C), lambda i: (i, 0, 0, 0)),
        ],
        out_specs=[
            pl.BlockSpec((BS, V, C, C), lambda i: (i, 0, 0, 0)),
            pl.BlockSpec((BS, 1, C), lambda i: (i, 0, 0)),
        ],
        out_shape=[
            jax.ShapeDtypeStruct((V, V, C, C), jnp.float32),
            jax.ShapeDtypeStruct((V, 1, C), jnp.float32),
        ],
        compiler_params=pltpu.CompilerParams(
            dimension_semantics=("arbitrary",),
        ),
    )(pc, counts)
    return out.reshape(-1), rs.reshape(-1)
